# baseline (speedup 1.0000x reference)
import functools

import jax
import jax.numpy as jnp
from jax import lax
from jax.experimental import pallas as pl
from jax.experimental.pallas import tpu as pltpu
from jax.experimental.pallas import tpu_sc as plsc

N = 10000
E = 320000
NPAD = 10240
NC, NS = 2, 16
NW = NC * NS
EPAD = 327680
EPT = EPAD // NW
K = 80
TOTB = EPAD // K
NBLK = TOTB // NW
PIECE = 64
BODY = 8
DG = 8
B_HEAVY = 3 * PIECE
B_LIGHT = PIECE
HEAVY_CID = 1
RPT = NPAD // NS
R = 1024
GRID = NPAD // R

_mesh = plsc.VectorSubcoreMesh(core_axis_name="c", subcore_axis_name="s")


def _deg_body(src_hbm, dst_hbm, ones_hbm, zeros_hbm, out_hbm,
              idx, ones_v, acc, sem):
    cid = lax.axis_index("c")
    sid = lax.axis_index("s")
    wid = cid * NS + sid
    sl = pl.ds(sid * RPT, RPT)
    pltpu.sync_copy(ones_hbm, ones_v)
    for phase in range(2):
        idx_hbm = (src_hbm, dst_hbm)[phase]
        pltpu.sync_copy(idx_hbm.at[pl.ds(wid * NBLK, NBLK)], idx)
        pltpu.sync_copy(zeros_hbm, acc.at[sl])
        plsc.subcore_barrier()

        def blk(j, carry):
            descs = [
                pltpu.async_copy(ones_v, acc.at[idx.at[j * DG + b]],
                                 sem, add=True)
                for b in range(DG)
            ]
            for d_ in descs:
                d_.wait()
            return carry

        lax.fori_loop(0, NBLK // DG, blk, 0)
        plsc.subcore_barrier()
        pltpu.sync_copy(acc.at[sl], out_hbm.at[cid, phase, sl])
        if phase == 0:
            plsc.subcore_barrier()


_deg_call = pl.kernel(
    _deg_body,
    out_type=jax.ShapeDtypeStruct((NC, 2, NPAD, 128), jnp.float32),
    mesh=_mesh,
    scratch_types=[
        pltpu.VMEM((NBLK, K), jnp.int32),
        pltpu.VMEM((K, 128), jnp.float32),
        pltpu.VMEM_SHARED((NPAD, 128), jnp.float32),
        pltpu.SemaphoreType.DMA,
    ],
)


def _make_agg(C):
    def body(*refs):
        tables = refs[:C]
        src_hbm, dst_hbm, zeros_hbm, out_hbm = refs[C:C + 4]
        sidx, didx, rows, acc = refs[C + 4:C + 8]
        semg = refs[C + 8:C + 11]
        sems = refs[C + 11:C + 14]
        cid = lax.axis_index("c")
        sid = lax.axis_index("s")
        wid = cid * NS + sid
        sl = pl.ds(sid * RPT, RPT)

        for c in range(C):
            pltpu.sync_copy(zeros_hbm, acc.at[sl])
            plsc.subcore_barrier()
            table = tables[c]

            def fire_g(blk, p):
                return pltpu.async_copy(table.at[sidx.at[blk]],
                                        rows.at[p], semg[p])

            def fire_s(blk, p):
                return pltpu.async_copy(rows.at[p], acc.at[didx.at[blk]],
                                        sems[p], add=True)

            def run8(j, carry):
                base = j * BODY
                g = {p: fire_g(base + p, p) for p in range(3)}
                s = {}
                for b in range(BODY):
                    g[b].wait()
                    s[b] = fire_s(base + b, b % 3)
                    if b >= 1:
                        s[b - 1].wait()
                        if b + 2 < BODY:
                            g[b + 2] = fire_g(base + b + 2, (b + 2) % 3)
                s[BODY - 1].wait()
                return carry

            def do_pieces(first_block, npiece):
                for p in range(npiece):
                    start = first_block + p * PIECE
                    pltpu.sync_copy(src_hbm.at[pl.ds(start, PIECE)], sidx)
                    pltpu.sync_copy(dst_hbm.at[pl.ds(start, PIECE)], didx)
                    lax.fori_loop(0, PIECE // BODY, run8, 0)

            @pl.when(cid == HEAVY_CID)
            def _():
                do_pieces(sid * B_HEAVY, B_HEAVY // PIECE)

            @pl.when(cid != HEAVY_CID)
            def _():
                do_pieces(NS * B_HEAVY + sid * B_LIGHT, B_LIGHT // PIECE)
            plsc.subcore_barrier()
            pltpu.sync_copy(acc.at[sl], out_hbm.at[cid, c, sl])
            if c != C - 1:
                plsc.subcore_barrier()

    return pl.kernel(
        body,
        out_type=jax.ShapeDtypeStruct((NC, C, NPAD, 128), jnp.float32),
        mesh=_mesh,
        scratch_types=[
            pltpu.VMEM((PIECE, K), jnp.int32),
            pltpu.VMEM((PIECE, K), jnp.int32),
            pltpu.VMEM((3, K, 128), jnp.float32),
            pltpu.VMEM_SHARED((NPAD, 128), jnp.float32),
            pltpu.SemaphoreType.DMA,
            pltpu.SemaphoreType.DMA,
            pltpu.SemaphoreType.DMA,
            pltpu.SemaphoreType.DMA,
            pltpu.SemaphoreType.DMA,
            pltpu.SemaphoreType.DMA,
        ],
    )


_agg1 = _make_agg(1)
_agg4 = _make_agg(4)
_agg2 = _make_agg(2)


def _prep_body(deg_ref, x_ref, nout_ref, nin_ref, c1_ref):
    dego = deg_ref[0, 0] + deg_ref[1, 0]
    degi = deg_ref[0, 1] + deg_ref[1, 1]
    no = lax.rsqrt(jnp.maximum(dego, 1.0))
    ni = lax.rsqrt(jnp.maximum(degi, 1.0))
    nout_ref[...] = no[:, :16]
    nin_ref[...] = ni[:, :16]
    c1_ref[...] = x_ref[...] * no[:, :1]


_prep_call = pl.pallas_call(
    _prep_body,
    grid=(GRID,),
    in_specs=[
        pl.BlockSpec((NC, 2, R, 128), lambda i: (0, 0, i, 0)),
        pl.BlockSpec((R, 128), lambda i: (i, 0)),
    ],
    out_specs=[
        pl.BlockSpec((R, 16), lambda i: (i, 0)),
        pl.BlockSpec((R, 16), lambda i: (i, 0)),
        pl.BlockSpec((R, 128), lambda i: (i, 0)),
    ],
    out_shape=[
        jax.ShapeDtypeStruct((NPAD, 16), jnp.float32),
        jax.ShapeDtypeStruct((NPAD, 16), jnp.float32),
        jax.ShapeDtypeStruct((NPAD, 128), jnp.float32),
    ],
)


def _layer1_body(r_ref, nin_ref, nout_ref, w_ref, out_ref):
    agg = (r_ref[0, 0] + r_ref[1, 0]) * nin_ref[:, :1]
    z = jnp.dot(agg, w_ref[...], preferred_element_type=jnp.float32)
    t = jnp.maximum(z, 0.0) * nout_ref[:, :1]
    for c in range(4):
        out_ref[c] = t[:, c * 128:(c + 1) * 128]


_layer1_call = pl.pallas_call(
    _layer1_body,
    grid=(GRID,),
    in_specs=[
        pl.BlockSpec((NC, 1, R, 128), lambda i: (0, 0, i, 0)),
        pl.BlockSpec((R, 16), lambda i: (i, 0)),
        pl.BlockSpec((R, 16), lambda i: (i, 0)),
        pl.BlockSpec((128, 512), lambda i: (0, 0)),
    ],
    out_specs=pl.BlockSpec((4, R, 128), lambda i: (0, i, 0)),
    out_shape=jax.ShapeDtypeStruct((4, NPAD, 128), jnp.float32),
)


def _layer23_body(r_ref, nin_ref, nout_ref, w2_ref, w3_ref, out_ref):
    h = jnp.concatenate(
        [r_ref[0, c] + r_ref[1, c] for c in range(4)], axis=1
    ) * nin_ref[:, :1]
    z = jnp.maximum(jnp.dot(h, w2_ref[...], preferred_element_type=jnp.float32), 0.0)
    g = jnp.dot(z, w3_ref[...], preferred_element_type=jnp.float32)
    t = g * nout_ref[:, :1]
    for c in range(2):
        out_ref[c] = t[:, c * 128:(c + 1) * 128]


_layer23_call = pl.pallas_call(
    _layer23_body,
    grid=(GRID,),
    in_specs=[
        pl.BlockSpec((NC, 4, R, 128), lambda i: (0, 0, i, 0)),
        pl.BlockSpec((R, 16), lambda i: (i, 0)),
        pl.BlockSpec((R, 16), lambda i: (i, 0)),
        pl.BlockSpec((512, 512), lambda i: (0, 0)),
        pl.BlockSpec((512, 256), lambda i: (0, 0)),
    ],
    out_specs=pl.BlockSpec((2, R, 128), lambda i: (0, i, 0)),
    out_shape=jax.ShapeDtypeStruct((2, NPAD, 128), jnp.float32),
)


def _final_body(r_ref, nin_ref, out_ref):
    out_ref[...] = jnp.concatenate(
        [r_ref[0, c] + r_ref[1, c] for c in range(2)], axis=1
    ) * nin_ref[:, :1]


_final_call = pl.pallas_call(
    _final_body,
    grid=(GRID,),
    in_specs=[
        pl.BlockSpec((NC, 2, R, 128), lambda i: (0, 0, i, 0)),
        pl.BlockSpec((R, 16), lambda i: (i, 0)),
    ],
    out_specs=pl.BlockSpec((R, 256), lambda i: (i, 0)),
    out_shape=jax.ShapeDtypeStruct((NPAD, 256), jnp.float32),
)


@jax.jit
def kernel(x, edge_index, W1, W2, W3):
    src = edge_index[0].astype(jnp.int32)
    dst = edge_index[1].astype(jnp.int32)
    pad = jnp.full((EPAD - E,), N, jnp.int32)
    src_p = jnp.concatenate([src, pad]).reshape(TOTB, K)
    dst_p = jnp.concatenate([dst, pad]).reshape(TOTB, K)
    x_pad = jnp.pad(x, ((0, NPAD - N), (0, 0)))
    ones128 = jnp.ones((K, 128), jnp.float32)
    zeros128 = jnp.zeros((RPT, 128), jnp.float32)

    deg = _deg_call(src_p, dst_p, ones128, zeros128)
    nout16, nin16, c1 = _prep_call(deg, x_pad)
    r1 = _agg1(c1, src_p, dst_p, zeros128)
    c2 = _layer1_call(r1, nin16, nout16, W1)
    r2 = _agg4(c2[0], c2[1], c2[2], c2[3], src_p, dst_p, zeros128)
    c3 = _layer23_call(r2, nin16, nout16, W2, W3)
    r3 = _agg2(c3[0], c3[1], src_p, dst_p, zeros128)
    out = _final_call(r3, nin16)
    return out[:N]


# speedup vs baseline: 2.6730x; 2.6730x over previous
import functools

import jax
import jax.numpy as jnp
from jax import lax
from jax.experimental import pallas as pl
from jax.experimental.pallas import tpu as pltpu
from jax.experimental.pallas import tpu_sc as plsc

N = 10000
E = 320000
NPAD = 10240
NC, NS = 2, 16
NW = NC * NS
EPAD = 327680
EPT = EPAD // NW
K = 80
TOTB = EPAD // K
NBLK = TOTB // NW
PIECE = 64
BODY = 8
DG = 8
B_HEAVY = 2 * PIECE
B_LIGHT = 2 * PIECE
HEAVY_CID = 0
RPT = NPAD // NS
R = 1024
GRID = NPAD // R

_mesh = plsc.VectorSubcoreMesh(core_axis_name="c", subcore_axis_name="s")


def _deg_body(src_hbm, dst_hbm, ones_hbm, zeros_hbm, out_hbm,
              idx, ones_v, acc, sem):
    cid = lax.axis_index("c")
    sid = lax.axis_index("s")
    wid = cid * NS + sid
    sl = pl.ds(sid * RPT, RPT)
    pltpu.sync_copy(ones_hbm, ones_v)
    for phase in range(2):
        idx_hbm = (src_hbm, dst_hbm)[phase]
        pltpu.sync_copy(idx_hbm.at[pl.ds(wid * NBLK, NBLK)], idx)
        pltpu.sync_copy(zeros_hbm, acc.at[sl])
        plsc.subcore_barrier()

        def blk(j, carry):
            descs = [
                pltpu.async_copy(ones_v, acc.at[idx.at[j * DG + b]],
                                 sem, add=True)
                for b in range(DG)
            ]
            for d_ in descs:
                d_.wait()
            return carry

        lax.fori_loop(0, NBLK // DG, blk, 0)
        plsc.subcore_barrier()
        pltpu.sync_copy(acc.at[sl], out_hbm.at[cid, phase, sl])
        if phase == 0:
            plsc.subcore_barrier()


_deg_call = pl.kernel(
    _deg_body,
    out_type=jax.ShapeDtypeStruct((NC, 2, NPAD, 128), jnp.float32),
    mesh=_mesh,
    scratch_types=[
        pltpu.VMEM((NBLK, K), jnp.int32),
        pltpu.VMEM((K, 128), jnp.float32),
        pltpu.VMEM_SHARED((NPAD, 128), jnp.float32),
        pltpu.SemaphoreType.DMA,
    ],
)


def _make_agg(C):
    def body(*refs):
        tables = refs[:C]
        src_hbm, dst_hbm, zeros_hbm, out_hbm = refs[C:C + 4]
        sidx, didx, rows, acc = refs[C + 4:C + 8]
        semg = refs[C + 8:C + 11]
        sems = refs[C + 11:C + 14]
        cid = lax.axis_index("c")
        sid = lax.axis_index("s")
        wid = cid * NS + sid
        sl = pl.ds(sid * RPT, RPT)

        for c in range(C):
            pltpu.sync_copy(zeros_hbm, acc.at[sl])
            plsc.subcore_barrier()
            table = tables[c]

            def fire_g(blk, p):
                return pltpu.async_copy(table.at[sidx.at[blk]],
                                        rows.at[p], semg[p])

            def fire_s(blk, p):
                return pltpu.async_copy(rows.at[p], acc.at[didx.at[blk]],
                                        sems[p], add=True)

            def run8(j, carry):
                base = j * BODY
                g = {p: fire_g(base + p, p) for p in range(3)}
                s = {}
                for b in range(BODY):
                    g[b].wait()
                    s[b] = fire_s(base + b, b % 3)
                    if b >= 1:
                        s[b - 1].wait()
                        if b + 2 < BODY:
                            g[b + 2] = fire_g(base + b + 2, (b + 2) % 3)
                s[BODY - 1].wait()
                return carry

            def do_pieces(first_block, npiece):
                for p in range(npiece):
                    start = first_block + p * PIECE
                    pltpu.sync_copy(src_hbm.at[pl.ds(start, PIECE)], sidx)
                    pltpu.sync_copy(dst_hbm.at[pl.ds(start, PIECE)], didx)
                    lax.fori_loop(0, PIECE // BODY, run8, 0)

            @pl.when(cid == HEAVY_CID)
            def _():
                do_pieces(sid * B_HEAVY, B_HEAVY // PIECE)

            @pl.when(cid != HEAVY_CID)
            def _():
                do_pieces(NS * B_HEAVY + sid * B_LIGHT, B_LIGHT // PIECE)
            plsc.subcore_barrier()
            pltpu.sync_copy(acc.at[sl], out_hbm.at[cid, c, sl])
            if c != C - 1:
                plsc.subcore_barrier()

    return pl.kernel(
        body,
        out_type=jax.ShapeDtypeStruct((NC, C, NPAD, 128), jnp.float32),
        mesh=_mesh,
        scratch_types=[
            pltpu.VMEM((PIECE, K), jnp.int32),
            pltpu.VMEM((PIECE, K), jnp.int32),
            pltpu.VMEM((3, K, 128), jnp.float32),
            pltpu.VMEM_SHARED((NPAD, 128), jnp.float32),
            pltpu.SemaphoreType.DMA,
            pltpu.SemaphoreType.DMA,
            pltpu.SemaphoreType.DMA,
            pltpu.SemaphoreType.DMA,
            pltpu.SemaphoreType.DMA,
            pltpu.SemaphoreType.DMA,
        ],
    )


_agg1 = _make_agg(1)
_agg4 = _make_agg(4)
_agg2 = _make_agg(2)


def _prep_body(deg_ref, x_ref, nout_ref, nin_ref, c1_ref):
    dego = deg_ref[0, 0] + deg_ref[1, 0]
    degi = deg_ref[0, 1] + deg_ref[1, 1]
    no = lax.rsqrt(jnp.maximum(dego, 1.0))
    ni = lax.rsqrt(jnp.maximum(degi, 1.0))
    nout_ref[...] = no[:, :16]
    nin_ref[...] = ni[:, :16]
    c1_ref[...] = x_ref[...] * no[:, :1]


_prep_call = pl.pallas_call(
    _prep_body,
    grid=(GRID,),
    in_specs=[
        pl.BlockSpec((NC, 2, R, 128), lambda i: (0, 0, i, 0)),
        pl.BlockSpec((R, 128), lambda i: (i, 0)),
    ],
    out_specs=[
        pl.BlockSpec((R, 16), lambda i: (i, 0)),
        pl.BlockSpec((R, 16), lambda i: (i, 0)),
        pl.BlockSpec((R, 128), lambda i: (i, 0)),
    ],
    out_shape=[
        jax.ShapeDtypeStruct((NPAD, 16), jnp.float32),
        jax.ShapeDtypeStruct((NPAD, 16), jnp.float32),
        jax.ShapeDtypeStruct((NPAD, 128), jnp.float32),
    ],
)


def _layer1_body(r_ref, nin_ref, nout_ref, w_ref, out_ref):
    agg = (r_ref[0, 0] + r_ref[1, 0]) * nin_ref[:, :1]
    z = jnp.dot(agg, w_ref[...], preferred_element_type=jnp.float32)
    t = jnp.maximum(z, 0.0) * nout_ref[:, :1]
    for c in range(4):
        out_ref[c] = t[:, c * 128:(c + 1) * 128]


_layer1_call = pl.pallas_call(
    _layer1_body,
    grid=(GRID,),
    in_specs=[
        pl.BlockSpec((NC, 1, R, 128), lambda i: (0, 0, i, 0)),
        pl.BlockSpec((R, 16), lambda i: (i, 0)),
        pl.BlockSpec((R, 16), lambda i: (i, 0)),
        pl.BlockSpec((128, 512), lambda i: (0, 0)),
    ],
    out_specs=pl.BlockSpec((4, R, 128), lambda i: (0, i, 0)),
    out_shape=jax.ShapeDtypeStruct((4, NPAD, 128), jnp.float32),
)


def _layer23_body(r_ref, nin_ref, nout_ref, w2_ref, w3_ref, out_ref):
    h = jnp.concatenate(
        [r_ref[0, c] + r_ref[1, c] for c in range(4)], axis=1
    ) * nin_ref[:, :1]
    z = jnp.maximum(jnp.dot(h, w2_ref[...], preferred_element_type=jnp.float32), 0.0)
    g = jnp.dot(z, w3_ref[...], preferred_element_type=jnp.float32)
    t = g * nout_ref[:, :1]
    for c in range(2):
        out_ref[c] = t[:, c * 128:(c + 1) * 128]


_layer23_call = pl.pallas_call(
    _layer23_body,
    grid=(GRID,),
    in_specs=[
        pl.BlockSpec((NC, 4, R, 128), lambda i: (0, 0, i, 0)),
        pl.BlockSpec((R, 16), lambda i: (i, 0)),
        pl.BlockSpec((R, 16), lambda i: (i, 0)),
        pl.BlockSpec((512, 512), lambda i: (0, 0)),
        pl.BlockSpec((512, 256), lambda i: (0, 0)),
    ],
    out_specs=pl.BlockSpec((2, R, 128), lambda i: (0, i, 0)),
    out_shape=jax.ShapeDtypeStruct((2, NPAD, 128), jnp.float32),
)


def _final_body(r_ref, nin_ref, out_ref):
    out_ref[...] = jnp.concatenate(
        [r_ref[0, c] + r_ref[1, c] for c in range(2)], axis=1
    ) * nin_ref[:, :1]


_final_call = pl.pallas_call(
    _final_body,
    grid=(GRID,),
    in_specs=[
        pl.BlockSpec((NC, 2, R, 128), lambda i: (0, 0, i, 0)),
        pl.BlockSpec((R, 16), lambda i: (i, 0)),
    ],
    out_specs=pl.BlockSpec((R, 256), lambda i: (i, 0)),
    out_shape=jax.ShapeDtypeStruct((NPAD, 256), jnp.float32),
)


@jax.jit
def kernel(x, edge_index, W1, W2, W3):
    src = edge_index[0].astype(jnp.int32)
    dst = edge_index[1].astype(jnp.int32)
    pad = N + jnp.arange(EPAD - E, dtype=jnp.int32) % (NPAD - N)
    src_p = jnp.concatenate([src, pad]).reshape(TOTB, K)
    dst_p = jnp.concatenate([dst, pad]).reshape(TOTB, K)
    x_pad = jnp.pad(x, ((0, NPAD - N), (0, 0)))
    ones128 = jnp.ones((K, 128), jnp.float32)
    zeros128 = jnp.zeros((RPT, 128), jnp.float32)

    deg = _deg_call(src_p, dst_p, ones128, zeros128)
    nout16, nin16, c1 = _prep_call(deg, x_pad)
    r1 = _agg1(c1, src_p, dst_p, zeros128)
    c2 = _layer1_call(r1, nin16, nout16, W1)
    r2 = _agg4(c2[0], c2[1], c2[2], c2[3], src_p, dst_p, zeros128)
    c3 = _layer23_call(r2, nin16, nout16, W2, W3)
    r3 = _agg2(c3[0], c3[1], src_p, dst_p, zeros128)
    out = _final_call(r3, nin16)
    return out[:N]
